# baseline (speedup 1.0000x reference)
import functools
import math

import jax
import jax.numpy as jnp
from jax import lax
from jax.experimental import pallas as pl
from jax.experimental.pallas import tpu as pltpu
from jax.experimental.pallas import tpu_sc as plsc

P = 10000
M = 10000
NB = 16
D = 128
H = 8
DH = D // H
DFF = 256
F = D * 3
FFW = DFF * 3

NW = 32
CHUNK = 128
NCHUNK = 20
CH_A = 24
CH_B = 2 * NCHUNK - CH_A
HALFN = 16 * (CH_A + CH_B) * CHUNK
NSLICE = 2
PH = P // NSLICE

BP = 200

_HI = lax.Precision.HIGHEST
_BF = jnp.bfloat16


def _kv_body(mem_ref, wk_ref, wv_ref, kv_ref):
    m16 = mem_ref[...].astype(_BF)
    kv_ref[:, :F] = jnp.dot(m16, wk_ref[...], preferred_element_type=jnp.float32)
    kv_ref[:, F:] = jnp.dot(m16, wv_ref[...], preferred_element_type=jnp.float32)


def _kv_projections(memory, WkKT, WvKT):
    nblk = M // 1000
    return pl.pallas_call(
        _kv_body,
        grid=(nblk,),
        in_specs=[
            pl.BlockSpec((1000, F), lambda i: (i, 0)),
            pl.BlockSpec((F, F), lambda i: (0, 0)),
            pl.BlockSpec((F, F), lambda i: (0, 0)),
        ],
        out_specs=pl.BlockSpec((1000, 2 * F), lambda i: (i, 0)),
        out_shape=jax.ShapeDtypeStruct((M, 2 * F), jnp.float32),
    )(memory, WkKT, WvKT)


def _q_body(tgt_ref, wq_ref, qf_ref):
    qf_ref[...] = jnp.dot(tgt_ref[...].astype(_BF), wq_ref[...],
                          preferred_element_type=jnp.float32)


def _q_projection(tgt, WqKT):
    nblk = P // 1000
    return pl.pallas_call(
        _q_body,
        grid=(nblk,),
        in_specs=[
            pl.BlockSpec((1000, F), lambda i: (i, 0)),
            pl.BlockSpec((F, F), lambda i: (0, 0)),
        ],
        out_specs=pl.BlockSpec((1000, F), lambda i: (i, 0)),
        out_shape=jax.ShapeDtypeStruct((P, F), jnp.float32),
    )(tgt, WqKT)


def _gather_kv(kvf, idx_pad):
    mesh = plsc.VectorSubcoreMesh(core_axis_name="c", subcore_axis_name="s")

    @functools.partial(
        pl.kernel,
        mesh=mesh,
        out_type=jax.ShapeDtypeStruct((HALFN, 2 * F), jnp.float32),
        scratch_types=[
            pltpu.VMEM((CH_A, CHUNK), jnp.int32),
            pltpu.VMEM((CH_B, CHUNK), jnp.int32),
            pltpu.VMEM((CHUNK, 2 * F), jnp.float32),
            pltpu.SemaphoreType.DMA,
        ],
    )
    def gather(kvf_hbm, idx_hbm, kvg_hbm, idx_va, idx_vb, kvbuf, sem):
        c = lax.axis_index("c")
        s = lax.axis_index("s")

        def run(idx_v, nch, chunk0):
            pltpu.sync_copy(idx_hbm.at[pl.ds(chunk0, nch)], idx_v)

            def body(t, carry):
                pltpu.async_copy(kvf_hbm.at[idx_v.at[t]], kvbuf, sem).wait()
                base = (chunk0 + t) * CHUNK
                pltpu.sync_copy(kvbuf, kvg_hbm.at[pl.ds(base, CHUNK)])
                return carry

            lax.fori_loop(0, nch, body, 0)

        @pl.when(c == 0)
        def _():
            run(idx_va, CH_A, s * CH_A)

        @pl.when(c == 1)
        def _():
            run(idx_vb, CH_B, 16 * CH_A + s * CH_B)

    return gather(kvf, idx_pad)


def _attn_ff_math(tgtb, qfb, kg3, vg3, shn3, dist3,
                  S, ST, E3, TD, TDT, TDff, TDffT,
                  Wg1T, bg1, Wg2T, bg2, WoKT, gn1, bn1,
                  Wf1KT, WdirKT, Wf2KT, gn2, bn2):
    f32 = jnp.float32
    q3 = qfb[:, None, :]
    doth = jnp.einsum('pnf,fh->pnh', kg3 * q3, S,
                      preferred_element_type=f32, precision=_HI)
    shrep = jnp.einsum('pni,if->pnf', shn3, E3,
                       preferred_element_type=f32, precision=_HI)
    pos = jnp.einsum('pnf,fd->pnd', (kg3 - q3) * shrep, TD,
                     preferred_element_type=f32, precision=_HI)
    g = jnp.maximum(jnp.einsum('pnd,dh->pnh', pos.astype(_BF), Wg1T,
                               preferred_element_type=f32) + bg1, 0.0)
    g = jnp.einsum('pnh,hg->png', g.astype(_BF), Wg2T,
                   preferred_element_type=f32) + bg2
    logits = (doth + g + dist3) * (1.0 / math.sqrt(DH))
    logits = logits - jnp.max(logits, axis=1, keepdims=True)
    e = jnp.exp(logits)
    att = e / jnp.sum(e, axis=1, keepdims=True)
    attf = jnp.einsum('pnh,hf->pnf', att, ST,
                      preferred_element_type=f32, precision=_HI)
    t2 = jnp.sum(attf * vg3, axis=1)
    t2 = jnp.dot(t2.astype(_BF), WoKT, preferred_element_type=f32)
    t = tgtb + t2

    def vnln(x, gamma, beta):
        sq = jnp.dot(x * x, TD, preferred_element_type=f32, precision=_HI)
        n = jnp.sqrt(sq + 1e-5)
        mu = jnp.mean(n, axis=-1, keepdims=True)
        var = jnp.mean((n - mu) ** 2, axis=-1, keepdims=True)
        ln = (n - mu) / jnp.sqrt(var + 1e-5) * gamma + beta
        return x * jnp.dot(ln / n, TDT, preferred_element_type=f32, precision=_HI)

    t = vnln(t, gn1, bn1)
    h = jnp.dot(t.astype(_BF), Wf1KT, preferred_element_type=f32)
    dvec = jnp.dot(h.astype(_BF), WdirKT, preferred_element_type=f32)
    dp = jnp.dot(h * dvec, TDff, preferred_element_type=f32, precision=_HI)
    den = jnp.dot(dvec * dvec, TDff, preferred_element_type=f32, precision=_HI) + 1e-6
    coef = jnp.where(dp >= 0.0, 0.0, dp / den)
    h = h - jnp.dot(coef, TDffT, preferred_element_type=f32, precision=_HI) * dvec
    t2 = jnp.dot(h.astype(_BF), Wf2KT, preferred_element_type=f32)
    t = t + t2
    t = vnln(t, gn2, bn2)
    return t


def _attn_ff_body(tgt_ref, qf_ref, kv_ref, shn_ref, dist_ref,
                  S_ref, ST_ref, E3_ref, TD_ref, TDT_ref, TDff_ref, TDffT_ref,
                  Wg1T_ref, bg1_ref, Wg2T_ref, bg2_ref, WoKT_ref, gn1_ref, bn1_ref,
                  Wf1KT_ref, WdirKT_ref, Wf2KT_ref, gn2_ref, bn2_ref, out_ref):
    out_ref[...] = _attn_ff_math(
        tgt_ref[...], qf_ref[...], kv_ref[:, :, :F], kv_ref[:, :, F:], shn_ref[...],
        dist_ref[...], S_ref[...], ST_ref[...], E3_ref[...], TD_ref[...],
        TDT_ref[...], TDff_ref[...], TDffT_ref[...], Wg1T_ref[...],
        bg1_ref[...], Wg2T_ref[...], bg2_ref[...], WoKT_ref[...],
        gn1_ref[...], bn1_ref[...], Wf1KT_ref[...], WdirKT_ref[...],
        Wf2KT_ref[...], gn2_ref[...], bn2_ref[...])


def _attn_ff(tgt, qf, kv3, shn3, dist3, consts):
    nblk = PH // BP
    full2 = lambda a: pl.BlockSpec(a.shape, lambda i: (0,) * a.ndim)
    in_specs = [
        pl.BlockSpec((BP, F), lambda i: (i, 0)),
        pl.BlockSpec((BP, F), lambda i: (i, 0)),
        pl.BlockSpec((BP, NB, 2 * F), lambda i: (i, 0, 0)),
        pl.BlockSpec((BP, NB, 3), lambda i: (i, 0, 0)),
        pl.BlockSpec((BP, NB, H), lambda i: (i, 0, 0)),
    ] + [full2(c) for c in consts]
    return pl.pallas_call(
        _attn_ff_body,
        grid=(nblk,),
        in_specs=in_specs,
        out_specs=pl.BlockSpec((BP, F), lambda i: (i, 0)),
        out_shape=jax.ShapeDtypeStruct((PH, F), jnp.float32),
    )(tgt, qf, kv3, shn3, dist3, *consts)


def kernel(tgt, memory, index_pair, cnt1, cnt2, sh, dist_atten,
           Wq, Wk, Wv, Wo, Wg1, bg1, Wg2, bg2, gn1, bn1, gn2, bn2,
           Wf1, Wdir, Wf2):
    f32 = jnp.float32
    I3 = jnp.eye(3, dtype=f32)

    WqKT = jnp.kron(Wq, I3).T.astype(_BF)
    WkKT = jnp.kron(Wk, I3).T.astype(_BF)
    WvKT = jnp.kron(Wv, I3).T.astype(_BF)
    WoKT = jnp.kron(Wo, I3).T.astype(_BF)
    Wf1KT = jnp.kron(Wf1, I3).T.astype(_BF)
    WdirKT = jnp.kron(Wdir, I3).T.astype(_BF)
    Wf2KT = jnp.kron(Wf2, I3).T.astype(_BF)

    fidx = jnp.arange(F)
    S = (fidx[:, None] // (3 * DH) == jnp.arange(H)[None, :]).astype(f32)
    ST = S.T
    E3 = (jnp.arange(3)[:, None] == (fidx % 3)[None, :]).astype(f32)
    TD = ((fidx[:, None] // 3) == jnp.arange(D)[None, :]).astype(f32)
    TDT = TD.T
    gidx = jnp.arange(FFW)
    TDff = ((gidx[:, None] // 3) == jnp.arange(DFF)[None, :]).astype(f32)
    TDffT = TDff.T

    consts = [S, ST, E3, TD, TDT, TDff, TDffT,
              Wg1.T.astype(_BF), bg1.reshape(1, 1, H), Wg2.T.astype(_BF),
              bg2.reshape(1, 1, H), WoKT, gn1.reshape(1, D), bn1.reshape(1, D),
              Wf1KT, WdirKT, Wf2KT, gn2.reshape(1, D), bn2.reshape(1, D)]

    kvf = _kv_projections(memory, WkKT, WvKT)
    qf = _q_projection(tgt, WqKT)

    idx_flat = index_pair.reshape(-1).astype(jnp.int32)
    outs = []
    for hid in range(NSLICE):
        idx_h = idx_flat[hid * PH * NB:(hid + 1) * PH * NB]
        idx_pad = jnp.concatenate(
            [idx_h, jnp.zeros((HALFN - PH * NB,), jnp.int32)]
        ).reshape(HALFN // CHUNK, CHUNK)
        kvg = _gather_kv(kvf, idx_pad)
        kv3 = kvg.reshape(HALFN // NB, NB, 2 * F)
        sl = slice(hid * PH, (hid + 1) * PH)
        outs.append(_attn_ff(tgt[sl], qf[sl], kv3,
                             sh[sl, 1:, :], dist_atten[sl], consts))
    return jnp.concatenate(outs, axis=0)


# speedup vs baseline: 1.0222x; 1.0222x over previous
import functools
import math

import jax
import jax.numpy as jnp
from jax import lax
from jax.experimental import pallas as pl
from jax.experimental.pallas import tpu as pltpu
from jax.experimental.pallas import tpu_sc as plsc

P = 10000
M = 10000
NB = 16
D = 128
H = 8
DH = D // H
DFF = 256
F = D * 3
FFW = DFF * 3

NW = 32
CHUNK = 128
NCHUNK = 20
ROWS_PER_W = CHUNK * NCHUNK
HALFN = NW * ROWS_PER_W
NSLICE = 2
PH = P // NSLICE

BP = 200

_HI = lax.Precision.HIGHEST
_BF = jnp.bfloat16


def _kv_body(mem_ref, wk_ref, wv_ref, kv_ref):
    m16 = mem_ref[...].astype(_BF)
    kv_ref[:, :F] = jnp.dot(m16, wk_ref[...], preferred_element_type=jnp.float32)
    kv_ref[:, F:] = jnp.dot(m16, wv_ref[...], preferred_element_type=jnp.float32)


def _kv_projections(memory, WkKT, WvKT):
    nblk = M // 1000
    return pl.pallas_call(
        _kv_body,
        grid=(nblk,),
        in_specs=[
            pl.BlockSpec((1000, F), lambda i: (i, 0)),
            pl.BlockSpec((F, F), lambda i: (0, 0)),
            pl.BlockSpec((F, F), lambda i: (0, 0)),
        ],
        out_specs=pl.BlockSpec((1000, 2 * F), lambda i: (i, 0)),
        out_shape=jax.ShapeDtypeStruct((M, 2 * F), jnp.float32),
    )(memory, WkKT, WvKT)


def _q_body(tgt_ref, wq_ref, qf_ref):
    qf_ref[...] = jnp.dot(tgt_ref[...].astype(_BF), wq_ref[...],
                          preferred_element_type=jnp.float32)


def _q_projection(tgt, WqKT):
    nblk = P // 1000
    return pl.pallas_call(
        _q_body,
        grid=(nblk,),
        in_specs=[
            pl.BlockSpec((1000, F), lambda i: (i, 0)),
            pl.BlockSpec((F, F), lambda i: (0, 0)),
        ],
        out_specs=pl.BlockSpec((1000, F), lambda i: (i, 0)),
        out_shape=jax.ShapeDtypeStruct((P, F), jnp.float32),
    )(tgt, WqKT)


def _gather_kv(kvf, idx_pad):
    mesh = plsc.VectorSubcoreMesh(core_axis_name="c", subcore_axis_name="s")

    @functools.partial(
        pl.kernel,
        mesh=mesh,
        out_type=jax.ShapeDtypeStruct((HALFN, 2 * F), jnp.float32),
        scratch_types=[
            pltpu.VMEM((NCHUNK, CHUNK), jnp.int32),
            pltpu.VMEM((CHUNK, 2 * F), jnp.float32),
            pltpu.SemaphoreType.DMA,
        ],
    )
    def gather(kvf_hbm, idx_hbm, kvg_hbm, idx_v, kvbuf, sem):
        wid = lax.axis_index("s") * 2 + lax.axis_index("c")
        pltpu.sync_copy(idx_hbm.at[wid], idx_v)

        def body(t, carry):
            pltpu.async_copy(kvf_hbm.at[idx_v.at[t]], kvbuf, sem).wait()
            base = wid * ROWS_PER_W + t * CHUNK
            pltpu.sync_copy(kvbuf, kvg_hbm.at[pl.ds(base, CHUNK)])
            return carry

        lax.fori_loop(0, NCHUNK, body, 0)

    return gather(kvf, idx_pad)


def _attn_ff_math(tgtb, qfb, kg3, vg3, shn3, dist3,
                  S, ST, E3, TD, TDT, TDff, TDffT,
                  Wg1T, bg1, Wg2T, bg2, WoKT, gn1, bn1,
                  Wf1KT, WdirKT, Wf2KT, gn2, bn2):
    f32 = jnp.float32
    BPn = tgtb.shape[0]
    R = BPn * NB
    kg2 = kg3.reshape(R, F)
    vg2 = vg3.reshape(R, F)
    qrep = jnp.broadcast_to(qfb[:, None, :], (BPn, NB, F)).reshape(R, F)
    doth = jnp.dot(kg2 * qrep, S, preferred_element_type=f32, precision=_HI)
    shrep = jnp.dot(shn3.reshape(R, 3), E3,
                    preferred_element_type=f32, precision=_HI)
    pos = jnp.dot((kg2 - qrep) * shrep, TD,
                  preferred_element_type=f32, precision=_HI)
    g = jnp.maximum(jnp.dot(pos.astype(_BF), Wg1T,
                            preferred_element_type=f32) + bg1, 0.0)
    g = jnp.dot(g.astype(_BF), Wg2T, preferred_element_type=f32) + bg2
    logits = (doth + g + dist3.reshape(R, H)) * (1.0 / math.sqrt(DH))
    l3 = logits.reshape(BPn, NB, H)
    l3 = l3 - jnp.max(l3, axis=1, keepdims=True)
    e = jnp.exp(l3)
    att = e / jnp.sum(e, axis=1, keepdims=True)
    attf = jnp.dot(att.reshape(R, H), ST,
                   preferred_element_type=f32, precision=_HI)
    t2 = jnp.sum((attf * vg2).reshape(BPn, NB, F), axis=1)
    t2 = jnp.dot(t2.astype(_BF), WoKT, preferred_element_type=f32)
    t = tgtb + t2

    def vnln(x, gamma, beta):
        sq = jnp.dot(x * x, TD, preferred_element_type=f32, precision=_HI)
        n = jnp.sqrt(sq + 1e-5)
        mu = jnp.mean(n, axis=-1, keepdims=True)
        var = jnp.mean((n - mu) ** 2, axis=-1, keepdims=True)
        ln = (n - mu) / jnp.sqrt(var + 1e-5) * gamma + beta
        return x * jnp.dot(ln / n, TDT, preferred_element_type=f32, precision=_HI)

    t = vnln(t, gn1, bn1)
    h = jnp.dot(t.astype(_BF), Wf1KT, preferred_element_type=f32)
    dvec = jnp.dot(h.astype(_BF), WdirKT, preferred_element_type=f32)
    dp = jnp.dot(h * dvec, TDff, preferred_element_type=f32, precision=_HI)
    den = jnp.dot(dvec * dvec, TDff, preferred_element_type=f32, precision=_HI) + 1e-6
    coef = jnp.where(dp >= 0.0, 0.0, dp / den)
    h = h - jnp.dot(coef, TDffT, preferred_element_type=f32, precision=_HI) * dvec
    t2 = jnp.dot(h.astype(_BF), Wf2KT, preferred_element_type=f32)
    t = t + t2
    t = vnln(t, gn2, bn2)
    return t


def _attn_ff_body(tgt_ref, qf_ref, kv_ref, shn_ref, dist_ref,
                  S_ref, ST_ref, E3_ref, TD_ref, TDT_ref, TDff_ref, TDffT_ref,
                  Wg1T_ref, bg1_ref, Wg2T_ref, bg2_ref, WoKT_ref, gn1_ref, bn1_ref,
                  Wf1KT_ref, WdirKT_ref, Wf2KT_ref, gn2_ref, bn2_ref, out_ref):
    out_ref[...] = _attn_ff_math(
        tgt_ref[...], qf_ref[...], kv_ref[:, :, :F], kv_ref[:, :, F:], shn_ref[...],
        dist_ref[...], S_ref[...], ST_ref[...], E3_ref[...], TD_ref[...],
        TDT_ref[...], TDff_ref[...], TDffT_ref[...], Wg1T_ref[...],
        bg1_ref[...], Wg2T_ref[...], bg2_ref[...], WoKT_ref[...],
        gn1_ref[...], bn1_ref[...], Wf1KT_ref[...], WdirKT_ref[...],
        Wf2KT_ref[...], gn2_ref[...], bn2_ref[...])


def _attn_ff(tgt, qf, kv3, shn3, dist3, consts):
    nblk = PH // BP
    full2 = lambda a: pl.BlockSpec(a.shape, lambda i: (0,) * a.ndim)
    in_specs = [
        pl.BlockSpec((BP, F), lambda i: (i, 0)),
        pl.BlockSpec((BP, F), lambda i: (i, 0)),
        pl.BlockSpec((BP, NB, 2 * F), lambda i: (i, 0, 0)),
        pl.BlockSpec((BP, NB, 3), lambda i: (i, 0, 0)),
        pl.BlockSpec((BP, NB, H), lambda i: (i, 0, 0)),
    ] + [full2(c) for c in consts]
    return pl.pallas_call(
        _attn_ff_body,
        grid=(nblk,),
        in_specs=in_specs,
        out_specs=pl.BlockSpec((BP, F), lambda i: (i, 0)),
        out_shape=jax.ShapeDtypeStruct((PH, F), jnp.float32),
    )(tgt, qf, kv3, shn3, dist3, *consts)


def kernel(tgt, memory, index_pair, cnt1, cnt2, sh, dist_atten,
           Wq, Wk, Wv, Wo, Wg1, bg1, Wg2, bg2, gn1, bn1, gn2, bn2,
           Wf1, Wdir, Wf2):
    f32 = jnp.float32
    I3 = jnp.eye(3, dtype=f32)

    WqKT = jnp.kron(Wq, I3).T.astype(_BF)
    WkKT = jnp.kron(Wk, I3).T.astype(_BF)
    WvKT = jnp.kron(Wv, I3).T.astype(_BF)
    WoKT = jnp.kron(Wo, I3).T.astype(_BF)
    Wf1KT = jnp.kron(Wf1, I3).T.astype(_BF)
    WdirKT = jnp.kron(Wdir, I3).T.astype(_BF)
    Wf2KT = jnp.kron(Wf2, I3).T.astype(_BF)

    fidx = jnp.arange(F)
    S = (fidx[:, None] // (3 * DH) == jnp.arange(H)[None, :]).astype(f32)
    ST = S.T
    E3 = (jnp.arange(3)[:, None] == (fidx % 3)[None, :]).astype(f32)
    TD = ((fidx[:, None] // 3) == jnp.arange(D)[None, :]).astype(f32)
    TDT = TD.T
    gidx = jnp.arange(FFW)
    TDff = ((gidx[:, None] // 3) == jnp.arange(DFF)[None, :]).astype(f32)
    TDffT = TDff.T

    consts = [S, ST, E3, TD, TDT, TDff, TDffT,
              Wg1.T.astype(_BF), bg1.reshape(1, H), Wg2.T.astype(_BF),
              bg2.reshape(1, H), WoKT, gn1.reshape(1, D), bn1.reshape(1, D),
              Wf1KT, WdirKT, Wf2KT, gn2.reshape(1, D), bn2.reshape(1, D)]

    kvf = _kv_projections(memory, WkKT, WvKT)
    qf = _q_projection(tgt, WqKT)

    idx_flat = index_pair.reshape(-1).astype(jnp.int32)
    outs = []
    for hid in range(NSLICE):
        idx_h = idx_flat[hid * PH * NB:(hid + 1) * PH * NB]
        idx_pad = jnp.concatenate(
            [idx_h, jnp.zeros((HALFN - PH * NB,), jnp.int32)]
        ).reshape(NW, NCHUNK, CHUNK)
        kvg = _gather_kv(kvf, idx_pad)
        kv3 = kvg.reshape(HALFN // NB, NB, 2 * F)
        sl = slice(hid * PH, (hid + 1) * PH)
        outs.append(_attn_ff(tgt[sl], qf[sl], kv3,
                             sh[sl, 1:, :], dist_atten[sl], consts))
    return jnp.concatenate(outs, axis=0)


# speedup vs baseline: 1.5011x; 1.4685x over previous
import functools
import math

import jax
import jax.numpy as jnp
from jax import lax
from jax.experimental import pallas as pl
from jax.experimental.pallas import tpu as pltpu
from jax.experimental.pallas import tpu_sc as plsc

P = 10000
M = 10000
NB = 16
D = 128
H = 8
DH = D // H
DFF = 256
F = D * 3
FFW = DFF * 3

NW = 32
CHUNK = 128
NCHUNK = 20
ROWS_PER_W = CHUNK * NCHUNK
HALFN = NW * ROWS_PER_W
NSLICE = 2
PH = P // NSLICE

BP = 200

_HI = lax.Precision.HIGHEST
_BF = jnp.bfloat16


def _proj_body(mem_ref, tgt_ref, wk_ref, wv_ref, wq_ref, kv_ref, qf_ref):
    m16 = mem_ref[...].astype(_BF)
    kv_ref[:, :F] = jnp.dot(m16, wk_ref[...], preferred_element_type=jnp.float32)
    kv_ref[:, F:] = jnp.dot(m16, wv_ref[...], preferred_element_type=jnp.float32)
    qf_ref[...] = jnp.dot(tgt_ref[...].astype(_BF), wq_ref[...],
                          preferred_element_type=jnp.float32)


def _projections(memory, tgt, WkKT, WvKT, WqKT):
    nblk = M // 1000
    return pl.pallas_call(
        _proj_body,
        grid=(nblk,),
        in_specs=[
            pl.BlockSpec((1000, F), lambda i: (i, 0)),
            pl.BlockSpec((1000, F), lambda i: (i, 0)),
            pl.BlockSpec((F, F), lambda i: (0, 0)),
            pl.BlockSpec((F, F), lambda i: (0, 0)),
            pl.BlockSpec((F, F), lambda i: (0, 0)),
        ],
        out_specs=[
            pl.BlockSpec((1000, 2 * F), lambda i: (i, 0)),
            pl.BlockSpec((1000, F), lambda i: (i, 0)),
        ],
        out_shape=[jax.ShapeDtypeStruct((M, 2 * F), jnp.float32),
                   jax.ShapeDtypeStruct((P, F), jnp.float32)],
    )(memory, tgt, WkKT, WvKT, WqKT)


def _gather_kv(kvf, idx_pad):
    mesh = plsc.VectorSubcoreMesh(core_axis_name="c", subcore_axis_name="s")

    @functools.partial(
        pl.kernel,
        mesh=mesh,
        out_type=jax.ShapeDtypeStruct((HALFN, 2 * F), jnp.float32),
        scratch_types=[
            pltpu.VMEM((NCHUNK, CHUNK), jnp.int32),
            pltpu.VMEM((CHUNK, 2 * F), jnp.float32),
            pltpu.SemaphoreType.DMA,
        ],
    )
    def gather(kvf_hbm, idx_hbm, kvg_hbm, idx_v, kvbuf, sem):
        wid = lax.axis_index("s") * 2 + lax.axis_index("c")
        pltpu.sync_copy(idx_hbm.at[wid], idx_v)

        def body(t, carry):
            pltpu.async_copy(kvf_hbm.at[idx_v.at[t]], kvbuf, sem).wait()
            base = wid * ROWS_PER_W + t * CHUNK
            pltpu.sync_copy(kvbuf, kvg_hbm.at[pl.ds(base, CHUNK)])
            return carry

        lax.fori_loop(0, NCHUNK, body, 0)

    return gather(kvf, idx_pad)


def _xdot(x, sel16):
    f32 = jnp.float32
    x1 = x.astype(_BF)
    r1 = x - x1.astype(f32)
    x2 = r1.astype(_BF)
    x3 = (r1 - x2.astype(f32)).astype(_BF)
    return (jnp.dot(x1, sel16, preferred_element_type=f32)
            + jnp.dot(x2, sel16, preferred_element_type=f32)
            + jnp.dot(x3, sel16, preferred_element_type=f32))

def _attn_ff_math(tgtb, qfb, kg3, vg3, shn3, dist3,
                  S, ST, E3, TD, TDT, TDff, TDffT,
                  Wg1T, bg1, Wg2T, bg2, WoKT, gn1, bn1,
                  Wf1KT, WdirKT, Wf2KT, gn2, bn2):
    f32 = jnp.float32
    BPn = tgtb.shape[0]
    R = BPn * NB
    kg2 = kg3.reshape(R, F)
    vg2 = vg3.reshape(R, F)
    qrep = jnp.broadcast_to(qfb[:, None, :], (BPn, NB, F)).reshape(R, F)
    doth = _xdot(kg2 * qrep, S)
    shrep = _xdot(shn3.reshape(R, 3), E3)
    pos = _xdot((kg2 - qrep) * shrep, TD)
    g = jnp.maximum(jnp.dot(pos.astype(_BF), Wg1T,
                            preferred_element_type=f32) + bg1, 0.0)
    g = jnp.dot(g.astype(_BF), Wg2T, preferred_element_type=f32) + bg2
    logits = (doth + g + dist3.reshape(R, H)) * (1.0 / math.sqrt(DH))
    l3 = logits.reshape(BPn, NB, H)
    l3 = l3 - jnp.max(l3, axis=1, keepdims=True)
    e = jnp.exp(l3)
    att = e / jnp.sum(e, axis=1, keepdims=True)
    attf = _xdot(att.reshape(R, H), ST)
    t2 = jnp.sum((attf * vg2).reshape(BPn, NB, F), axis=1)
    t2 = jnp.dot(t2.astype(_BF), WoKT, preferred_element_type=f32)
    t = tgtb + t2

    def vnln(x, gamma, beta):
        sq = _xdot(x * x, TD)
        n = jnp.sqrt(sq + 1e-5)
        mu = jnp.mean(n, axis=-1, keepdims=True)
        var = jnp.mean((n - mu) ** 2, axis=-1, keepdims=True)
        ln = (n - mu) / jnp.sqrt(var + 1e-5) * gamma + beta
        return x * _xdot(ln / n, TDT)

    t = vnln(t, gn1, bn1)
    h = jnp.dot(t.astype(_BF), Wf1KT, preferred_element_type=f32)
    dvec = jnp.dot(h.astype(_BF), WdirKT, preferred_element_type=f32)
    dp = _xdot(h * dvec, TDff)
    den = _xdot(dvec * dvec, TDff) + 1e-6
    coef = jnp.where(dp >= 0.0, 0.0, dp / den)
    h = h - _xdot(coef, TDffT) * dvec
    t2 = jnp.dot(h.astype(_BF), Wf2KT, preferred_element_type=f32)
    t = t + t2
    t = vnln(t, gn2, bn2)
    return t


def _attn_ff_body(tgt_ref, qf_ref, kv_ref, shn_ref, dist_ref,
                  S_ref, ST_ref, E3_ref, TD_ref, TDT_ref, TDff_ref, TDffT_ref,
                  Wg1T_ref, bg1_ref, Wg2T_ref, bg2_ref, WoKT_ref, gn1_ref, bn1_ref,
                  Wf1KT_ref, WdirKT_ref, Wf2KT_ref, gn2_ref, bn2_ref, out_ref):
    out_ref[...] = _attn_ff_math(
        tgt_ref[...], qf_ref[...], kv_ref[:, :, :F], kv_ref[:, :, F:], shn_ref[...],
        dist_ref[...], S_ref[...], ST_ref[...], E3_ref[...], TD_ref[...],
        TDT_ref[...], TDff_ref[...], TDffT_ref[...], Wg1T_ref[...],
        bg1_ref[...], Wg2T_ref[...], bg2_ref[...], WoKT_ref[...],
        gn1_ref[...], bn1_ref[...], Wf1KT_ref[...], WdirKT_ref[...],
        Wf2KT_ref[...], gn2_ref[...], bn2_ref[...])


def _attn_ff(tgt, qf, kv3, shn3, dist3, consts):
    nblk = PH // BP
    full2 = lambda a: pl.BlockSpec(a.shape, lambda i: (0,) * a.ndim)
    in_specs = [
        pl.BlockSpec((BP, F), lambda i: (i, 0)),
        pl.BlockSpec((BP, F), lambda i: (i, 0)),
        pl.BlockSpec((BP, NB, 2 * F), lambda i: (i, 0, 0)),
        pl.BlockSpec((BP, NB, 3), lambda i: (i, 0, 0)),
        pl.BlockSpec((BP, NB, H), lambda i: (i, 0, 0)),
    ] + [full2(c) for c in consts]
    return pl.pallas_call(
        _attn_ff_body,
        grid=(nblk,),
        in_specs=in_specs,
        out_specs=pl.BlockSpec((BP, F), lambda i: (i, 0)),
        out_shape=jax.ShapeDtypeStruct((PH, F), jnp.float32),
    )(tgt, qf, kv3, shn3, dist3, *consts)


def kernel(tgt, memory, index_pair, cnt1, cnt2, sh, dist_atten,
           Wq, Wk, Wv, Wo, Wg1, bg1, Wg2, bg2, gn1, bn1, gn2, bn2,
           Wf1, Wdir, Wf2):
    f32 = jnp.float32
    I3 = jnp.eye(3, dtype=f32)

    WqKT = jnp.kron(Wq, I3).T.astype(_BF)
    WkKT = jnp.kron(Wk, I3).T.astype(_BF)
    WvKT = jnp.kron(Wv, I3).T.astype(_BF)
    WoKT = jnp.kron(Wo, I3).T.astype(_BF)
    Wf1KT = jnp.kron(Wf1, I3).T.astype(_BF)
    WdirKT = jnp.kron(Wdir, I3).T.astype(_BF)
    Wf2KT = jnp.kron(Wf2, I3).T.astype(_BF)

    fidx = jnp.arange(F)
    S = (fidx[:, None] // (3 * DH) == jnp.arange(H)[None, :]).astype(f32)
    ST = S.T
    E3 = (jnp.arange(3)[:, None] == (fidx % 3)[None, :]).astype(f32)
    TD = ((fidx[:, None] // 3) == jnp.arange(D)[None, :]).astype(f32)
    TDT = TD.T
    gidx = jnp.arange(FFW)
    TDff = ((gidx[:, None] // 3) == jnp.arange(DFF)[None, :]).astype(f32)
    TDffT = TDff.T

    consts = [S.astype(_BF), ST.astype(_BF), E3.astype(_BF), TD.astype(_BF),
              TDT.astype(_BF), TDff.astype(_BF), TDffT.astype(_BF),
              Wg1.T.astype(_BF), bg1.reshape(1, H), Wg2.T.astype(_BF),
              bg2.reshape(1, H), WoKT, gn1.reshape(1, D), bn1.reshape(1, D),
              Wf1KT, WdirKT, Wf2KT, gn2.reshape(1, D), bn2.reshape(1, D)]

    kvf, qf = _projections(memory, tgt, WkKT, WvKT, WqKT)

    idx_flat = index_pair.reshape(-1).astype(jnp.int32)
    outs = []
    for hid in range(NSLICE):
        idx_h = idx_flat[hid * PH * NB:(hid + 1) * PH * NB]
        idx_pad = jnp.concatenate(
            [idx_h, jnp.zeros((HALFN - PH * NB,), jnp.int32)]
        ).reshape(NW, NCHUNK, CHUNK)
        kvg = _gather_kv(kvf, idx_pad)
        kv3 = kvg.reshape(HALFN // NB, NB, 2 * F)
        sl = slice(hid * PH, (hid + 1) * PH)
        outs.append(_attn_ff(tgt[sl], qf[sl], kv3,
                             sh[sl, 1:, :], dist_atten[sl], consts))
    return jnp.concatenate(outs, axis=0)
